# Initial kernel scaffold; baseline (speedup 1.0000x reference)
import jax
import jax.numpy as jnp
from jax.experimental import pallas as pl


def kernel(x_target, x_context, lstm_t_Wih, lstm_t_Whh, lstm_t_bih, lstm_t_bhh, lstm_c_Wih, lstm_c_Whh, lstm_c_bih, lstm_c_bhh, dir_self_W, dir_self_b, dir_s2d_W, dir_s2d_b, dir_d2s_W, dir_d2s_b, ct_Wl, ct_bl, ct_Wr, lin_W, lin_b, edge_index_tt, edge_index_ct):
    raise NotImplementedError("write your pallas kernel here")


# speedup vs baseline: 3.6759x; 3.6759x over previous
import jax
import jax.numpy as jnp
from jax import lax
from jax.experimental import pallas as pl
from jax.experimental.pallas import tpu as pltpu
from jax.experimental.pallas import tpu_sc as plsc

H = 128
GW = 4 * H
NT = 10000
NC = 10000
SEQ = 8
ALPHA = 0.5
CHUNK = 128
LANES = 16
NTILE = 16
NCORE = 2
HALF = NT // NCORE
ACC_ROWS = 5120
ROWS_PER_TILE = ACC_ROWS // NTILE
IGN = 1 << 30
E_TT = 320000
E_CT = 160000
E0_PAD = 327680
E2_PAD = 163840
LSTM_R = 1000


def _lstm_body(x_ref, wih_ref, whh_ref, b_ref, out_ref):
    x = x_ref[...]
    wih = wih_ref[0]
    whh = whh_ref[0]
    b = b_ref[0]

    def gates(g):
        i = jax.nn.sigmoid(g[:, 0:H])
        f = jax.nn.sigmoid(g[:, H:2 * H])
        gg = jnp.tanh(g[:, 2 * H:3 * H])
        o = jax.nn.sigmoid(g[:, 3 * H:4 * H])
        return i, f, gg, o

    g = x[:, 0:1] * wih + b
    i, f, gg, o = gates(g)
    c = i * gg
    h = o * jnp.tanh(c)
    for t in range(1, SEQ):
        g = x[:, t:t + 1] * wih + b
        g = g + jnp.dot(h, whh, preferred_element_type=jnp.float32)
        i, f, gg, o = gates(g)
        c = f * c + i * gg
        h = o * jnp.tanh(c)
    out_ref[...] = h


def _run_lstm(x_all, wih_all, whh_all, b_all):
    n = x_all.shape[0]
    grid = n // LSTM_R
    return pl.pallas_call(
        _lstm_body,
        grid=(grid,),
        in_specs=[
            pl.BlockSpec((LSTM_R, SEQ), lambda i: (i, 0)),
            pl.BlockSpec((1, 1, GW), lambda i: (i // (grid // 2), 0, 0)),
            pl.BlockSpec((1, H, GW), lambda i: (i // (grid // 2), 0, 0)),
            pl.BlockSpec((1, 1, GW), lambda i: (i // (grid // 2), 0, 0)),
        ],
        out_specs=pl.BlockSpec((LSTM_R, H), lambda i: (i, 0)),
        out_shape=jax.ShapeDtypeStruct((n, H), jnp.float32),
    )(x_all, wih_all, whh_all, b_all)


def _sc_agg_body(table_h, s0_h, d0_h, s1_h, d1_h, s2_h, d2_h, zeros_h,
                 zflat_h, sums_h, cnts_h,
                 src_v, dst_v, rows_v, zero_v, hist_v, acc_sh, sem):
    cid = lax.axis_index("c")
    sid = lax.axis_index("s")
    base_row = sid * ROWS_PER_TILE
    pltpu.sync_copy(zeros_h, zero_v)

    def run(agg_i, srcs_h, dsts_h, ntile_chunks):
        pltpu.sync_copy(srcs_h.at[cid, pl.ds(sid * ntile_chunks, ntile_chunks)],
                        src_v.at[pl.ds(0, ntile_chunks)])
        pltpu.sync_copy(dsts_h.at[cid, pl.ds(sid * ntile_chunks, ntile_chunks)],
                        dst_v.at[pl.ds(0, ntile_chunks)])
        pltpu.sync_copy(zero_v, acc_sh.at[pl.ds(base_row, CHUNK)])
        pltpu.sync_copy(zero_v, acc_sh.at[pl.ds(base_row + CHUNK, CHUNK)])
        pltpu.sync_copy(zero_v.at[pl.ds(0, ROWS_PER_TILE - 2 * CHUNK)],
                        acc_sh.at[pl.ds(base_row + 2 * CHUNK,
                                        ROWS_PER_TILE - 2 * CHUNK)])
        pltpu.sync_copy(zflat_h, hist_v)
        plsc.subcore_barrier()

        ones16 = jnp.ones((LANES,), jnp.float32)

        def chunk(j, carry):
            pltpu.async_copy(
                table_h.at[plsc.Indices(src_v.at[j], ignored_value=IGN)],
                rows_v, sem).wait()
            pltpu.sync_copy(
                rows_v,
                acc_sh.at[plsc.Indices(dst_v.at[j], ignored_value=IGN)],
                add=True)
            for k in range(CHUNK // LANES):
                d = dst_v[j, pl.ds(k * LANES, LANES)]
                plsc.addupdate_scatter(hist_v, [d], ones16,
                                       mask=d < ACC_ROWS)
            return carry

        lax.fori_loop(0, ntile_chunks, chunk, 0)
        plsc.subcore_barrier()
        pltpu.sync_copy(acc_sh.at[pl.ds(base_row, CHUNK)],
                        sums_h.at[agg_i, cid, pl.ds(base_row, CHUNK)])
        pltpu.sync_copy(acc_sh.at[pl.ds(base_row + CHUNK, CHUNK)],
                        sums_h.at[agg_i, cid, pl.ds(base_row + CHUNK, CHUNK)])
        pltpu.sync_copy(
            acc_sh.at[pl.ds(base_row + 2 * CHUNK, ROWS_PER_TILE - 2 * CHUNK)],
            sums_h.at[agg_i, cid, pl.ds(base_row + 2 * CHUNK,
                                        ROWS_PER_TILE - 2 * CHUNK)])
        pltpu.sync_copy(hist_v, cnts_h.at[agg_i, cid, sid])

    run(0, s0_h, d0_h, E0_PAD // (NTILE * CHUNK))
    run(1, s1_h, d1_h, E0_PAD // (NTILE * CHUNK))
    run(2, s2_h, d2_h, E2_PAD // (NTILE * CHUNK))


def _run_sc_agg(table, s0, d0, s1, d1, s2, d2):
    zeros_chunk = jnp.zeros((CHUNK, H), jnp.float32)
    zeros_flat = jnp.zeros((ACC_ROWS,), jnp.float32)
    nch0 = E0_PAD // (NTILE * CHUNK)
    return pl.kernel(
        _sc_agg_body,
        out_type=(
            jax.ShapeDtypeStruct((3, NCORE, ACC_ROWS, H), jnp.float32),
            jax.ShapeDtypeStruct((3, NCORE, NTILE, ACC_ROWS), jnp.float32),
        ),
        mesh=plsc.VectorSubcoreMesh(core_axis_name="c", subcore_axis_name="s"),
        compiler_params=pltpu.CompilerParams(needs_layout_passes=False),
        scratch_types=[
            pltpu.VMEM((nch0, CHUNK), jnp.int32),
            pltpu.VMEM((nch0, CHUNK), jnp.int32),
            pltpu.VMEM((CHUNK, H), jnp.float32),
            pltpu.VMEM((CHUNK, H), jnp.float32),
            pltpu.VMEM((ACC_ROWS,), jnp.float32),
            pltpu.VMEM_SHARED((ACC_ROWS, H), jnp.float32),
            pltpu.SemaphoreType.DMA,
        ],
    )(table, s0, d0, s1, d1, s2, d2, zeros_chunk, zeros_flat)


def _epi_body(t_ref, sums_ref, cnts_ref, wh_ref, wa_ref, bc_ref, wl_ref,
              bl_ref, out_ref):
    ht = t_ref[...]
    pre = jnp.dot(ht, wh_ref[...], preferred_element_type=jnp.float32) + bc_ref[...]
    for a in range(3):
        s = sums_ref[a, 0]
        cnt = jnp.maximum(jnp.sum(cnts_ref[a, 0], axis=0), 1.0)
        pre = pre + jnp.dot(s / cnt, wa_ref[a],
                            preferred_element_type=jnp.float32)
    act = jnp.maximum(pre + ht, 0.0)
    out_ref[...] = (jnp.dot(act, wl_ref[...], preferred_element_type=jnp.float32)
                    + bl_ref[...])


def _run_epilogue(table, sums, cnts, w_h, w_agg, b_const, w_lin, b_lin):
    grid = NT // LSTM_R
    nb = HALF // LSTM_R
    return pl.pallas_call(
        _epi_body,
        grid=(grid,),
        in_specs=[
            pl.BlockSpec((LSTM_R, H), lambda i: (i, 0)),
            pl.BlockSpec((3, 1, LSTM_R, H), lambda i: (0, i // nb, i % nb, 0)),
            pl.BlockSpec((3, 1, NTILE, LSTM_R, 1),
                         lambda i: (0, i // nb, 0, i % nb, 0)),
            pl.BlockSpec((H, H), lambda i: (0, 0)),
            pl.BlockSpec((3, H, H), lambda i: (0, 0, 0)),
            pl.BlockSpec((1, H), lambda i: (0, 0)),
            pl.BlockSpec((H, H), lambda i: (0, 0)),
            pl.BlockSpec((1, H), lambda i: (0, 0)),
        ],
        out_specs=pl.BlockSpec((LSTM_R, H), lambda i: (i, 0)),
        out_shape=jax.ShapeDtypeStruct((NT, H), jnp.float32),
    )(table, sums, cnts, w_h, w_agg, b_const, w_lin, b_lin)


def _split_edges(s, d):
    ss, dd = [], []
    for c in range(NCORE):
        valid = (d >= c * HALF) & (d < (c + 1) * HALF)
        ss.append(jnp.where(valid, s, IGN))
        dd.append(jnp.where(valid, d - c * HALF, IGN))
    return (jnp.stack(ss).reshape(NCORE, -1, CHUNK),
            jnp.stack(dd).reshape(NCORE, -1, CHUNK))


def kernel(x_target, x_context, lstm_t_Wih, lstm_t_Whh, lstm_t_bih, lstm_t_bhh,
           lstm_c_Wih, lstm_c_Whh, lstm_c_bih, lstm_c_bhh,
           dir_self_W, dir_self_b, dir_s2d_W, dir_s2d_b, dir_d2s_W, dir_d2s_b,
           ct_Wl, ct_bl, ct_Wr, lin_W, lin_b, edge_index_tt, edge_index_ct):
    f32 = jnp.float32
    x_all = jnp.concatenate([x_target, x_context], axis=0).astype(f32)
    wih_all = jnp.stack([lstm_t_Wih.T, lstm_c_Wih.T])
    whh_all = jnp.stack([lstm_t_Whh.T, lstm_c_Whh.T])
    b_all = jnp.stack([(lstm_t_bih + lstm_t_bhh)[None, :],
                       (lstm_c_bih + lstm_c_bhh)[None, :]])
    table = _run_lstm(x_all, wih_all, whh_all, b_all)

    i32 = jnp.int32
    p0 = E0_PAD - E_TT
    p2 = E2_PAD - E_CT
    zpad0 = jnp.zeros((p0,), i32)
    npad0 = jnp.full((p0,), NT, i32)
    s0, d0 = _split_edges(jnp.concatenate([edge_index_tt[0], zpad0]),
                          jnp.concatenate([edge_index_tt[1], npad0]))
    s1, d1 = _split_edges(jnp.concatenate([edge_index_tt[1], zpad0]),
                          jnp.concatenate([edge_index_tt[0], npad0]))
    s2, d2 = _split_edges(jnp.concatenate([edge_index_ct[0] + NT,
                                           jnp.zeros((p2,), i32)]),
                          jnp.concatenate([edge_index_ct[1],
                                           jnp.full((p2,), NT, i32)]))

    sums, cnts = _run_sc_agg(table, s0, d0, s1, d1, s2, d2)
    cnts = cnts[..., None]

    w_h = ((dir_self_W.T + ct_Wr.T) * 0.5).astype(f32)
    w_agg = jnp.stack([
        ((1.0 - ALPHA) * 0.5) * dir_s2d_W.T,
        (ALPHA * 0.5) * dir_d2s_W.T,
        0.5 * ct_Wl.T,
    ]).astype(f32)
    b_const = ((dir_self_b + (1.0 - ALPHA) * dir_s2d_b + ALPHA * dir_d2s_b
                + ct_bl) * 0.5)[None, :].astype(f32)
    return _run_epilogue(table, sums, cnts, w_h, w_agg, b_const,
                         lin_W.T.astype(f32), lin_b[None, :].astype(f32))


# speedup vs baseline: 5.3830x; 1.4644x over previous
import jax
import jax.numpy as jnp
from jax import lax
from jax.experimental import pallas as pl
from jax.experimental.pallas import tpu as pltpu
from jax.experimental.pallas import tpu_sc as plsc

H = 128
GW = 4 * H
NT = 10000
NC = 10000
SEQ = 8
ALPHA = 0.5
CHUNK = 128
LANES = 16
NTILE = 16
NCORE = 2
HALF = NT // NCORE
ACC_ROWS = 5120
ROWS_PER_TILE = ACC_ROWS // NTILE
IGN = 1 << 30
E_TT = 320000
E_CT = 160000
E0_PAD = 327680
E2_PAD = 163840
LSTM_R = 1000


def _lstm_body(x_ref, wih_ref, whh_ref, b_ref, out_ref):
    x = x_ref[...]
    wih = wih_ref[0]
    whh = whh_ref[0]
    b = b_ref[0]

    def gates(g):
        i = jax.nn.sigmoid(g[:, 0:H])
        f = jax.nn.sigmoid(g[:, H:2 * H])
        gg = jnp.tanh(g[:, 2 * H:3 * H])
        o = jax.nn.sigmoid(g[:, 3 * H:4 * H])
        return i, f, gg, o

    g = x[:, 0:1] * wih + b
    i, f, gg, o = gates(g)
    c = i * gg
    h = o * jnp.tanh(c)
    for t in range(1, SEQ):
        g = x[:, t:t + 1] * wih + b
        g = g + jnp.dot(h, whh, preferred_element_type=jnp.float32)
        i, f, gg, o = gates(g)
        c = f * c + i * gg
        h = o * jnp.tanh(c)
    out_ref[...] = h


def _run_lstm(x_all, wih_all, whh_all, b_all):
    n = x_all.shape[0]
    grid = n // LSTM_R
    return pl.pallas_call(
        _lstm_body,
        grid=(grid,),
        in_specs=[
            pl.BlockSpec((LSTM_R, SEQ), lambda i: (i, 0)),
            pl.BlockSpec((1, 1, GW), lambda i: (i // (grid // 2), 0, 0)),
            pl.BlockSpec((1, H, GW), lambda i: (i // (grid // 2), 0, 0)),
            pl.BlockSpec((1, 1, GW), lambda i: (i // (grid // 2), 0, 0)),
        ],
        out_specs=pl.BlockSpec((LSTM_R, H), lambda i: (i, 0)),
        out_shape=jax.ShapeDtypeStruct((n, H), jnp.float32),
    )(x_all, wih_all, whh_all, b_all)


def _sc_agg_body(table_h, s0_h, d0_h, s1_h, d1_h, s2_h, d2_h, zeros_h,
                 zflat_h, sums_h, cnts_h,
                 src_v, dst_v, rows_a, rows_b, hist_v, acc_sh,
                 sem_a, sem_b):
    cid = lax.axis_index("c")
    sid = lax.axis_index("s")
    base_row = sid * ROWS_PER_TILE

    def run(agg_i, srcs_h, dsts_h, ntile_chunks):
        pltpu.sync_copy(srcs_h.at[cid, pl.ds(sid * ntile_chunks, ntile_chunks)],
                        src_v.at[pl.ds(0, ntile_chunks)])
        pltpu.sync_copy(dsts_h.at[cid, pl.ds(sid * ntile_chunks, ntile_chunks)],
                        dst_v.at[pl.ds(0, ntile_chunks)])
        pltpu.sync_copy(zeros_h, acc_sh.at[pl.ds(base_row, CHUNK)])
        pltpu.sync_copy(zeros_h, acc_sh.at[pl.ds(base_row + CHUNK, CHUNK)])
        pltpu.sync_copy(zeros_h.at[pl.ds(0, ROWS_PER_TILE - 2 * CHUNK)],
                        acc_sh.at[pl.ds(base_row + 2 * CHUNK,
                                        ROWS_PER_TILE - 2 * CHUNK)])
        pltpu.sync_copy(zflat_h, hist_v)
        plsc.subcore_barrier()

        ones16 = jnp.ones((LANES,), jnp.float32)

        def gather(j, buf, sem):
            return pltpu.async_copy(
                table_h.at[plsc.Indices(src_v.at[j], ignored_value=IGN)],
                buf, sem)

        def scatter(j, buf):
            pltpu.sync_copy(
                buf,
                acc_sh.at[plsc.Indices(dst_v.at[j], ignored_value=IGN)],
                add=True)
            for k in range(CHUNK // LANES):
                d = dst_v[j, pl.ds(k * LANES, LANES)]
                plsc.addupdate_scatter(hist_v, [d], ones16,
                                       mask=d < ACC_ROWS)

        def chunk2(i, carry):
            j = 2 * i
            desc_b = gather(j + 1, rows_b, sem_b)
            pltpu.make_async_copy(
                table_h.at[plsc.Indices(src_v.at[j], ignored_value=IGN)],
                rows_a, sem_a).wait()
            scatter(j, rows_a)

            @pl.when(j + 2 < ntile_chunks)
            def _():
                gather(j + 2, rows_a, sem_a)

            desc_b.wait()
            scatter(j + 1, rows_b)
            return carry

        gather(0, rows_a, sem_a)
        lax.fori_loop(0, ntile_chunks // 2, chunk2, 0)
        plsc.subcore_barrier()
        pltpu.sync_copy(acc_sh.at[pl.ds(base_row, CHUNK)],
                        sums_h.at[agg_i, cid, pl.ds(base_row, CHUNK)])
        pltpu.sync_copy(acc_sh.at[pl.ds(base_row + CHUNK, CHUNK)],
                        sums_h.at[agg_i, cid, pl.ds(base_row + CHUNK, CHUNK)])
        pltpu.sync_copy(
            acc_sh.at[pl.ds(base_row + 2 * CHUNK, ROWS_PER_TILE - 2 * CHUNK)],
            sums_h.at[agg_i, cid, pl.ds(base_row + 2 * CHUNK,
                                        ROWS_PER_TILE - 2 * CHUNK)])
        pltpu.sync_copy(hist_v, cnts_h.at[agg_i, cid, sid])

    run(0, s0_h, d0_h, E0_PAD // (NTILE * CHUNK))
    run(1, s1_h, d1_h, E0_PAD // (NTILE * CHUNK))
    run(2, s2_h, d2_h, E2_PAD // (NTILE * CHUNK))


def _run_sc_agg(table, s0, d0, s1, d1, s2, d2):
    zeros_chunk = jnp.zeros((CHUNK, H), jnp.float32)
    zeros_flat = jnp.zeros((ACC_ROWS,), jnp.float32)
    nch0 = E0_PAD // (NTILE * CHUNK)
    return pl.kernel(
        _sc_agg_body,
        out_type=(
            jax.ShapeDtypeStruct((3, NCORE, ACC_ROWS, H), jnp.float32),
            jax.ShapeDtypeStruct((3, NCORE, NTILE, ACC_ROWS), jnp.float32),
        ),
        mesh=plsc.VectorSubcoreMesh(core_axis_name="c", subcore_axis_name="s"),
        compiler_params=pltpu.CompilerParams(needs_layout_passes=False),
        scratch_types=[
            pltpu.VMEM((nch0, CHUNK), jnp.int32),
            pltpu.VMEM((nch0, CHUNK), jnp.int32),
            pltpu.VMEM((CHUNK, H), jnp.float32),
            pltpu.VMEM((CHUNK, H), jnp.float32),
            pltpu.VMEM((ACC_ROWS,), jnp.float32),
            pltpu.VMEM_SHARED((ACC_ROWS, H), jnp.float32),
            pltpu.SemaphoreType.DMA,
            pltpu.SemaphoreType.DMA,
        ],
    )(table, s0, d0, s1, d1, s2, d2, zeros_chunk, zeros_flat)


def _epi_body(t_ref, sums_ref, cnts_ref, wh_ref, wa_ref, bc_ref, wl_ref,
              bl_ref, out_ref):
    ht = t_ref[...]
    pre = jnp.dot(ht, wh_ref[...], preferred_element_type=jnp.float32) + bc_ref[...]
    for a in range(3):
        s = sums_ref[a, 0]
        cnt = jnp.maximum(jnp.sum(cnts_ref[a, 0], axis=0), 1.0)
        pre = pre + jnp.dot(s / cnt, wa_ref[a],
                            preferred_element_type=jnp.float32)
    act = jnp.maximum(pre + ht, 0.0)
    out_ref[...] = (jnp.dot(act, wl_ref[...], preferred_element_type=jnp.float32)
                    + bl_ref[...])


def _run_epilogue(table, sums, cnts, w_h, w_agg, b_const, w_lin, b_lin):
    grid = NT // LSTM_R
    nb = HALF // LSTM_R
    return pl.pallas_call(
        _epi_body,
        grid=(grid,),
        in_specs=[
            pl.BlockSpec((LSTM_R, H), lambda i: (i, 0)),
            pl.BlockSpec((3, 1, LSTM_R, H), lambda i: (0, i // nb, i % nb, 0)),
            pl.BlockSpec((3, 1, NTILE, LSTM_R, 1),
                         lambda i: (0, i // nb, 0, i % nb, 0)),
            pl.BlockSpec((H, H), lambda i: (0, 0)),
            pl.BlockSpec((3, H, H), lambda i: (0, 0, 0)),
            pl.BlockSpec((1, H), lambda i: (0, 0)),
            pl.BlockSpec((H, H), lambda i: (0, 0)),
            pl.BlockSpec((1, H), lambda i: (0, 0)),
        ],
        out_specs=pl.BlockSpec((LSTM_R, H), lambda i: (i, 0)),
        out_shape=jax.ShapeDtypeStruct((NT, H), jnp.float32),
    )(table, sums, cnts, w_h, w_agg, b_const, w_lin, b_lin)


def _split_edges(s, d):
    ss, dd = [], []
    for c in range(NCORE):
        valid = (d >= c * HALF) & (d < (c + 1) * HALF)
        ss.append(jnp.where(valid, s, IGN))
        dd.append(jnp.where(valid, d - c * HALF, IGN))
    return (jnp.stack(ss).reshape(NCORE, -1, CHUNK),
            jnp.stack(dd).reshape(NCORE, -1, CHUNK))


def kernel(x_target, x_context, lstm_t_Wih, lstm_t_Whh, lstm_t_bih, lstm_t_bhh,
           lstm_c_Wih, lstm_c_Whh, lstm_c_bih, lstm_c_bhh,
           dir_self_W, dir_self_b, dir_s2d_W, dir_s2d_b, dir_d2s_W, dir_d2s_b,
           ct_Wl, ct_bl, ct_Wr, lin_W, lin_b, edge_index_tt, edge_index_ct):
    f32 = jnp.float32
    x_all = jnp.concatenate([x_target, x_context], axis=0).astype(f32)
    wih_all = jnp.stack([lstm_t_Wih.T, lstm_c_Wih.T])
    whh_all = jnp.stack([lstm_t_Whh.T, lstm_c_Whh.T])
    b_all = jnp.stack([(lstm_t_bih + lstm_t_bhh)[None, :],
                       (lstm_c_bih + lstm_c_bhh)[None, :]])
    table = _run_lstm(x_all, wih_all, whh_all, b_all)

    i32 = jnp.int32
    p0 = E0_PAD - E_TT
    p2 = E2_PAD - E_CT
    zpad0 = jnp.zeros((p0,), i32)
    npad0 = jnp.full((p0,), NT, i32)
    s0, d0 = _split_edges(jnp.concatenate([edge_index_tt[0], zpad0]),
                          jnp.concatenate([edge_index_tt[1], npad0]))
    s1, d1 = _split_edges(jnp.concatenate([edge_index_tt[1], zpad0]),
                          jnp.concatenate([edge_index_tt[0], npad0]))
    s2, d2 = _split_edges(jnp.concatenate([edge_index_ct[0] + NT,
                                           jnp.zeros((p2,), i32)]),
                          jnp.concatenate([edge_index_ct[1],
                                           jnp.full((p2,), NT, i32)]))

    sums, cnts = _run_sc_agg(table, s0, d0, s1, d1, s2, d2)
    cnts = cnts[..., None]

    w_h = ((dir_self_W.T + ct_Wr.T) * 0.5).astype(f32)
    w_agg = jnp.stack([
        ((1.0 - ALPHA) * 0.5) * dir_s2d_W.T,
        (ALPHA * 0.5) * dir_d2s_W.T,
        0.5 * ct_Wl.T,
    ]).astype(f32)
    b_const = ((dir_self_b + (1.0 - ALPHA) * dir_s2d_b + ALPHA * dir_d2s_b
                + ct_bl) * 0.5)[None, :].astype(f32)
    return _run_epilogue(table, sums, cnts, w_h, w_agg, b_const,
                         lin_W.T.astype(f32), lin_b[None, :].astype(f32))


# speedup vs baseline: 7.0482x; 1.3093x over previous
import jax
import jax.numpy as jnp
from jax import lax
from jax.experimental import pallas as pl
from jax.experimental.pallas import tpu as pltpu
from jax.experimental.pallas import tpu_sc as plsc

H = 128
GW = 4 * H
NT = 10000
NC = 10000
SEQ = 8
ALPHA = 0.5
CHUNK = 128
LANES = 16
NTILE = 16
NCORE = 2
ACC_ROWS = 5120
HALF = ACC_ROWS
ROWS_PER_TILE = ACC_ROWS // NTILE
IGN = 1 << 30
E_TT = 320000
E_CT = 160000
E0_PAD = 327680
E2_PAD = 163840
LSTM_R = 1000
EPI_R = 1024


def _lstm_body(x_ref, wih_ref, whh_ref, b_ref, out_ref):
    x = x_ref[...]
    wih = wih_ref[0]
    whh = whh_ref[0]
    b = b_ref[0]

    def gates(g):
        i = jax.nn.sigmoid(g[:, 0:H])
        f = jax.nn.sigmoid(g[:, H:2 * H])
        gg = jnp.tanh(g[:, 2 * H:3 * H])
        o = jax.nn.sigmoid(g[:, 3 * H:4 * H])
        return i, f, gg, o

    g = x[:, 0:1] * wih + b
    i, f, gg, o = gates(g)
    c = i * gg
    h = o * jnp.tanh(c)
    for t in range(1, SEQ):
        g = x[:, t:t + 1] * wih + b
        g = g + jnp.dot(h, whh, preferred_element_type=jnp.float32)
        i, f, gg, o = gates(g)
        c = f * c + i * gg
        h = o * jnp.tanh(c)
    out_ref[...] = h


def _run_lstm(x_all, wih_all, whh_all, b_all):
    n = x_all.shape[0]
    grid = n // LSTM_R
    return pl.pallas_call(
        _lstm_body,
        grid=(grid,),
        in_specs=[
            pl.BlockSpec((LSTM_R, SEQ), lambda i: (i, 0)),
            pl.BlockSpec((1, 1, GW), lambda i: (i // (grid // 2), 0, 0)),
            pl.BlockSpec((1, H, GW), lambda i: (i // (grid // 2), 0, 0)),
            pl.BlockSpec((1, 1, GW), lambda i: (i // (grid // 2), 0, 0)),
        ],
        out_specs=pl.BlockSpec((LSTM_R, H), lambda i: (i, 0)),
        out_shape=jax.ShapeDtypeStruct((n, H), jnp.float32),
    )(x_all, wih_all, whh_all, b_all)


def _sc_agg_body(table_h, s0_h, d0_h, s1_h, d1_h, s2_h, d2_h, zeros_h,
                 zflat_h, sums_h, cnts_h,
                 src_v, dst_v, rows_a, rows_b, hist_v, acc_sh,
                 sem_a, sem_b):
    cid = lax.axis_index("c")
    sid = lax.axis_index("s")
    base_row = sid * ROWS_PER_TILE

    def run(agg_i, srcs_h, dsts_h, ntile_chunks):
        pltpu.sync_copy(srcs_h.at[cid, pl.ds(sid * ntile_chunks, ntile_chunks)],
                        src_v.at[pl.ds(0, ntile_chunks)])
        pltpu.sync_copy(dsts_h.at[cid, pl.ds(sid * ntile_chunks, ntile_chunks)],
                        dst_v.at[pl.ds(0, ntile_chunks)])
        pltpu.sync_copy(zeros_h, acc_sh.at[pl.ds(base_row, CHUNK)])
        pltpu.sync_copy(zeros_h, acc_sh.at[pl.ds(base_row + CHUNK, CHUNK)])
        pltpu.sync_copy(zeros_h.at[pl.ds(0, ROWS_PER_TILE - 2 * CHUNK)],
                        acc_sh.at[pl.ds(base_row + 2 * CHUNK,
                                        ROWS_PER_TILE - 2 * CHUNK)])
        pltpu.sync_copy(zflat_h, hist_v)
        plsc.subcore_barrier()

        ones16 = jnp.ones((LANES,), jnp.float32)

        def gather(j, buf, sem):
            return pltpu.async_copy(
                table_h.at[plsc.Indices(src_v.at[j], ignored_value=IGN)],
                buf, sem)

        def scatter(j, buf):
            pltpu.sync_copy(
                buf,
                acc_sh.at[plsc.Indices(dst_v.at[j], ignored_value=IGN)],
                add=True)
            for k in range(CHUNK // LANES):
                d = dst_v[j, pl.ds(k * LANES, LANES)]
                plsc.addupdate_scatter(hist_v, [d], ones16,
                                       mask=d < ACC_ROWS)

        def chunk2(i, carry):
            j = 2 * i
            desc_b = gather(j + 1, rows_b, sem_b)
            pltpu.make_async_copy(
                table_h.at[plsc.Indices(src_v.at[j], ignored_value=IGN)],
                rows_a, sem_a).wait()
            scatter(j, rows_a)

            @pl.when(j + 2 < ntile_chunks)
            def _():
                gather(j + 2, rows_a, sem_a)

            desc_b.wait()
            scatter(j + 1, rows_b)
            return carry

        gather(0, rows_a, sem_a)
        lax.fori_loop(0, ntile_chunks // 2, chunk2, 0)
        plsc.subcore_barrier()
        pltpu.sync_copy(acc_sh.at[pl.ds(base_row, CHUNK)],
                        sums_h.at[agg_i, cid, pl.ds(base_row, CHUNK)])
        pltpu.sync_copy(acc_sh.at[pl.ds(base_row + CHUNK, CHUNK)],
                        sums_h.at[agg_i, cid, pl.ds(base_row + CHUNK, CHUNK)])
        pltpu.sync_copy(
            acc_sh.at[pl.ds(base_row + 2 * CHUNK, ROWS_PER_TILE - 2 * CHUNK)],
            sums_h.at[agg_i, cid, pl.ds(base_row + 2 * CHUNK,
                                        ROWS_PER_TILE - 2 * CHUNK)])
        pltpu.sync_copy(hist_v, cnts_h.at[agg_i, cid, sid])

    run(0, s0_h, d0_h, E0_PAD // (NTILE * CHUNK))
    run(1, s1_h, d1_h, E0_PAD // (NTILE * CHUNK))
    run(2, s2_h, d2_h, E2_PAD // (NTILE * CHUNK))


def _run_sc_agg(table, s0, d0, s1, d1, s2, d2):
    zeros_chunk = jnp.zeros((CHUNK, H), jnp.float32)
    zeros_flat = jnp.zeros((ACC_ROWS,), jnp.float32)
    nch0 = E0_PAD // (NTILE * CHUNK)
    return pl.kernel(
        _sc_agg_body,
        out_type=(
            jax.ShapeDtypeStruct((3, NCORE, ACC_ROWS, H), jnp.float32),
            jax.ShapeDtypeStruct((3, NCORE, NTILE, ACC_ROWS), jnp.float32),
        ),
        mesh=plsc.VectorSubcoreMesh(core_axis_name="c", subcore_axis_name="s"),
        compiler_params=pltpu.CompilerParams(needs_layout_passes=False),
        scratch_types=[
            pltpu.VMEM((nch0, CHUNK), jnp.int32),
            pltpu.VMEM((nch0, CHUNK), jnp.int32),
            pltpu.VMEM((CHUNK, H), jnp.float32),
            pltpu.VMEM((CHUNK, H), jnp.float32),
            pltpu.VMEM((ACC_ROWS,), jnp.float32),
            pltpu.VMEM_SHARED((ACC_ROWS, H), jnp.float32),
            pltpu.SemaphoreType.DMA,
            pltpu.SemaphoreType.DMA,
        ],
    )(table, s0, d0, s1, d1, s2, d2, zeros_chunk, zeros_flat)


def _epi_body(t_ref, sums_ref, cnts_ref, wh_ref, wa_ref, bc_ref, wl_ref,
              bl_ref, out_ref):
    ht = t_ref[...]
    pre = jnp.dot(ht, wh_ref[...], preferred_element_type=jnp.float32) + bc_ref[...]
    for a in range(3):
        s = sums_ref[a, 0]
        cnt = jnp.maximum(jnp.sum(cnts_ref[a, 0], axis=0), 1.0)
        pre = pre + jnp.dot(s / cnt[:, None], wa_ref[a],
                            preferred_element_type=jnp.float32)
    act = jnp.maximum(pre + ht, 0.0)
    out_ref[...] = (jnp.dot(act, wl_ref[...], preferred_element_type=jnp.float32)
                    + bl_ref[...])


def _run_epilogue(table, sums, cnts, w_h, w_agg, b_const, w_lin, b_lin):
    grid = (NT + EPI_R - 1) // EPI_R
    nb = HALF // EPI_R
    return pl.pallas_call(
        _epi_body,
        grid=(grid,),
        in_specs=[
            pl.BlockSpec((EPI_R, H), lambda i: (i, 0)),
            pl.BlockSpec((3, 1, EPI_R, H), lambda i: (0, i // nb, i % nb, 0)),
            pl.BlockSpec((3, 1, NTILE, EPI_R),
                         lambda i: (0, i // nb, 0, i % nb)),
            pl.BlockSpec((H, H), lambda i: (0, 0)),
            pl.BlockSpec((3, H, H), lambda i: (0, 0, 0)),
            pl.BlockSpec((1, H), lambda i: (0, 0)),
            pl.BlockSpec((H, H), lambda i: (0, 0)),
            pl.BlockSpec((1, H), lambda i: (0, 0)),
        ],
        out_specs=pl.BlockSpec((EPI_R, H), lambda i: (i, 0)),
        out_shape=jax.ShapeDtypeStruct((NT, H), jnp.float32),
    )(table, sums, cnts, w_h, w_agg, b_const, w_lin, b_lin)


def _split_edges(s, d):
    ss, dd = [], []
    for c in range(NCORE):
        valid = (d >= c * HALF) & (d < (c + 1) * HALF)
        ss.append(jnp.where(valid, s, IGN))
        dd.append(jnp.where(valid, d - c * HALF, IGN))
    return (jnp.stack(ss).reshape(NCORE, -1, CHUNK),
            jnp.stack(dd).reshape(NCORE, -1, CHUNK))


def kernel(x_target, x_context, lstm_t_Wih, lstm_t_Whh, lstm_t_bih, lstm_t_bhh,
           lstm_c_Wih, lstm_c_Whh, lstm_c_bih, lstm_c_bhh,
           dir_self_W, dir_self_b, dir_s2d_W, dir_s2d_b, dir_d2s_W, dir_d2s_b,
           ct_Wl, ct_bl, ct_Wr, lin_W, lin_b, edge_index_tt, edge_index_ct):
    f32 = jnp.float32
    x_all = jnp.concatenate([x_target, x_context], axis=0).astype(f32)
    wih_all = jnp.stack([lstm_t_Wih.T, lstm_c_Wih.T])
    whh_all = jnp.stack([lstm_t_Whh.T, lstm_c_Whh.T])
    b_all = jnp.stack([(lstm_t_bih + lstm_t_bhh)[None, :],
                       (lstm_c_bih + lstm_c_bhh)[None, :]])
    table = _run_lstm(x_all, wih_all, whh_all, b_all)

    i32 = jnp.int32
    p0 = E0_PAD - E_TT
    p2 = E2_PAD - E_CT
    zpad0 = jnp.zeros((p0,), i32)
    npad0 = jnp.full((p0,), NCORE * HALF, i32)
    s0, d0 = _split_edges(jnp.concatenate([edge_index_tt[0], zpad0]),
                          jnp.concatenate([edge_index_tt[1], npad0]))
    s1, d1 = _split_edges(jnp.concatenate([edge_index_tt[1], zpad0]),
                          jnp.concatenate([edge_index_tt[0], npad0]))
    s2, d2 = _split_edges(jnp.concatenate([edge_index_ct[0] + NT,
                                           jnp.zeros((p2,), i32)]),
                          jnp.concatenate([edge_index_ct[1],
                                           jnp.full((p2,), NCORE * HALF, i32)]))

    sums, cnts = _run_sc_agg(table, s0, d0, s1, d1, s2, d2)

    w_h = ((dir_self_W.T + ct_Wr.T) * 0.5).astype(f32)
    w_agg = jnp.stack([
        ((1.0 - ALPHA) * 0.5) * dir_s2d_W.T,
        (ALPHA * 0.5) * dir_d2s_W.T,
        0.5 * ct_Wl.T,
    ]).astype(f32)
    b_const = ((dir_self_b + (1.0 - ALPHA) * dir_s2d_b + ALPHA * dir_d2s_b
                + ct_bl) * 0.5)[None, :].astype(f32)
    return _run_epilogue(table, sums, cnts, w_h, w_agg, b_const,
                         lin_W.T.astype(f32), lin_b[None, :].astype(f32))


# speedup vs baseline: 7.7272x; 1.0963x over previous
import functools

import jax
import jax.numpy as jnp
from jax import lax
from jax.experimental import pallas as pl
from jax.experimental.pallas import tpu as pltpu
from jax.experimental.pallas import tpu_sc as plsc

H = 128
GW = 4 * H
NT = 10000
NC = 10000
SEQ = 8
ALPHA = 0.5
CHUNK = 128
LANES = 16
NTILE = 16
NCORE = 2
ACC_ROWS = 5120
HALF = ACC_ROWS
ROWS_PER_TILE = ACC_ROWS // NTILE
IGN = 1 << 30
E_TT = 320000
E_CT = 160000
E0_PAD = 327680
E2_PAD = 163840
LSTM_R = 1000
EPI_R = 1024


def _lstm_body(x_ref, wih_ref, whh_ref, b_ref, out_ref):
    x = x_ref[...]
    wih = wih_ref[...]
    whh = whh_ref[...]
    b = b_ref[...]

    def gates(g):
        i = jax.nn.sigmoid(g[:, 0:H])
        f = jax.nn.sigmoid(g[:, H:2 * H])
        gg = jnp.tanh(g[:, 2 * H:3 * H])
        o = jax.nn.sigmoid(g[:, 3 * H:4 * H])
        return i, f, gg, o

    g = x[:, 0:1] * wih + b
    i, f, gg, o = gates(g)
    c = i * gg
    h = o * jnp.tanh(c)
    for t in range(1, SEQ):
        g = x[:, t:t + 1] * wih + b
        g = g + jnp.dot(h, whh, preferred_element_type=jnp.float32)
        i, f, gg, o = gates(g)
        c = f * c + i * gg
        h = o * jnp.tanh(c)
    out_ref[...] = h


def _run_lstm(x, wih, whh, bih, bhh):
    n = x.shape[0]
    return pl.pallas_call(
        _lstm_body,
        grid=(n // LSTM_R,),
        in_specs=[
            pl.BlockSpec((LSTM_R, SEQ), lambda i: (i, 0)),
            pl.BlockSpec((1, GW), lambda i: (0, 0)),
            pl.BlockSpec((H, GW), lambda i: (0, 0)),
            pl.BlockSpec((1, GW), lambda i: (0, 0)),
        ],
        out_specs=pl.BlockSpec((LSTM_R, H), lambda i: (i, 0)),
        out_shape=jax.ShapeDtypeStruct((n, H), jnp.float32),
    )(x.astype(jnp.float32), wih.T, whh.T, (bih + bhh)[None, :])


def _sc_agg_body(n_aggs, chunk_counts, *refs):
    table_h = refs[0]
    edges = refs[1:1 + 2 * n_aggs]
    (zeros_h, zflat_h, sums_h, cnts_h,
     src_v, dst_v, rows_a, rows_b, hist_v, acc_sh, sem_a, sem_b) = \
        refs[1 + 2 * n_aggs:]
    cid = lax.axis_index("c")
    sid = lax.axis_index("s")
    base_row = sid * ROWS_PER_TILE

    def run(agg_i, srcs_h, dsts_h, ntile_chunks):
        pltpu.sync_copy(srcs_h.at[cid, pl.ds(sid * ntile_chunks, ntile_chunks)],
                        src_v.at[pl.ds(0, ntile_chunks)])
        pltpu.sync_copy(dsts_h.at[cid, pl.ds(sid * ntile_chunks, ntile_chunks)],
                        dst_v.at[pl.ds(0, ntile_chunks)])
        pltpu.sync_copy(zeros_h, acc_sh.at[pl.ds(base_row, CHUNK)])
        pltpu.sync_copy(zeros_h, acc_sh.at[pl.ds(base_row + CHUNK, CHUNK)])
        pltpu.sync_copy(zeros_h.at[pl.ds(0, ROWS_PER_TILE - 2 * CHUNK)],
                        acc_sh.at[pl.ds(base_row + 2 * CHUNK,
                                        ROWS_PER_TILE - 2 * CHUNK)])
        pltpu.sync_copy(zflat_h, hist_v)
        plsc.subcore_barrier()

        ones16 = jnp.ones((LANES,), jnp.float32)

        def gather(j, buf, sem):
            return pltpu.async_copy(
                table_h.at[plsc.Indices(src_v.at[j], ignored_value=IGN)],
                buf, sem)

        def scatter(j, buf):
            pltpu.sync_copy(
                buf,
                acc_sh.at[plsc.Indices(dst_v.at[j], ignored_value=IGN)],
                add=True)
            for k in range(CHUNK // LANES):
                d = dst_v[j, pl.ds(k * LANES, LANES)]
                plsc.addupdate_scatter(hist_v, [d], ones16,
                                       mask=d < ACC_ROWS)

        def chunk2(i, carry):
            j = 2 * i
            desc_b = gather(j + 1, rows_b, sem_b)
            pltpu.make_async_copy(
                table_h.at[plsc.Indices(src_v.at[j], ignored_value=IGN)],
                rows_a, sem_a).wait()
            scatter(j, rows_a)

            @pl.when(j + 2 < ntile_chunks)
            def _():
                gather(j + 2, rows_a, sem_a)

            desc_b.wait()
            scatter(j + 1, rows_b)
            return carry

        gather(0, rows_a, sem_a)
        lax.fori_loop(0, ntile_chunks // 2, chunk2, 0)
        plsc.subcore_barrier()
        pltpu.sync_copy(acc_sh.at[pl.ds(base_row, CHUNK)],
                        sums_h.at[agg_i, cid, pl.ds(base_row, CHUNK)])
        pltpu.sync_copy(acc_sh.at[pl.ds(base_row + CHUNK, CHUNK)],
                        sums_h.at[agg_i, cid, pl.ds(base_row + CHUNK, CHUNK)])
        pltpu.sync_copy(
            acc_sh.at[pl.ds(base_row + 2 * CHUNK, ROWS_PER_TILE - 2 * CHUNK)],
            sums_h.at[agg_i, cid, pl.ds(base_row + 2 * CHUNK,
                                        ROWS_PER_TILE - 2 * CHUNK)])
        pltpu.sync_copy(hist_v, cnts_h.at[agg_i, cid, sid])

    for a in range(n_aggs):
        run(a, edges[2 * a], edges[2 * a + 1], chunk_counts[a])


def _run_sc_agg(table, edges, chunk_counts):
    n_aggs = len(chunk_counts)
    zeros_chunk = jnp.zeros((CHUNK, H), jnp.float32)
    zeros_flat = jnp.zeros((ACC_ROWS,), jnp.float32)
    nch = max(chunk_counts)
    return pl.kernel(
        functools.partial(_sc_agg_body, n_aggs, tuple(chunk_counts)),
        out_type=(
            jax.ShapeDtypeStruct((n_aggs, NCORE, ACC_ROWS, H), jnp.float32),
            jax.ShapeDtypeStruct((n_aggs, NCORE, NTILE, ACC_ROWS),
                                 jnp.float32),
        ),
        mesh=plsc.VectorSubcoreMesh(core_axis_name="c", subcore_axis_name="s"),
        compiler_params=pltpu.CompilerParams(needs_layout_passes=False),
        scratch_types=[
            pltpu.VMEM((nch, CHUNK), jnp.int32),
            pltpu.VMEM((nch, CHUNK), jnp.int32),
            pltpu.VMEM((CHUNK, H), jnp.float32),
            pltpu.VMEM((CHUNK, H), jnp.float32),
            pltpu.VMEM((ACC_ROWS,), jnp.float32),
            pltpu.VMEM_SHARED((ACC_ROWS, H), jnp.float32),
            pltpu.SemaphoreType.DMA,
            pltpu.SemaphoreType.DMA,
        ],
    )(table, *edges, zeros_chunk, zeros_flat)


def _epi_body(t_ref, stt_ref, ctt_ref, sct_ref, cct_ref,
              wh_ref, wa_ref, bc_ref, wl_ref, bl_ref, out_ref):
    ht = t_ref[...]
    pre = jnp.dot(ht, wh_ref[...], preferred_element_type=jnp.float32) + bc_ref[...]
    parts = [(stt_ref[0, 0], ctt_ref[0, 0]), (stt_ref[1, 0], ctt_ref[1, 0]),
             (sct_ref[0, 0], cct_ref[0, 0])]
    for a, (s, craw) in enumerate(parts):
        cnt = jnp.maximum(jnp.sum(craw, axis=0), 1.0)
        pre = pre + jnp.dot(s / cnt[:, None], wa_ref[a],
                            preferred_element_type=jnp.float32)
    act = jnp.maximum(pre + ht, 0.0)
    out_ref[...] = (jnp.dot(act, wl_ref[...], preferred_element_type=jnp.float32)
                    + bl_ref[...])


def _run_epilogue(table, sums_tt, cnts_tt, sums_ct, cnts_ct,
                  w_h, w_agg, b_const, w_lin, b_lin):
    grid = (NT + EPI_R - 1) // EPI_R
    nb = HALF // EPI_R
    return pl.pallas_call(
        _epi_body,
        grid=(grid,),
        in_specs=[
            pl.BlockSpec((EPI_R, H), lambda i: (i, 0)),
            pl.BlockSpec((2, 1, EPI_R, H), lambda i: (0, i // nb, i % nb, 0)),
            pl.BlockSpec((2, 1, NTILE, EPI_R),
                         lambda i: (0, i // nb, 0, i % nb)),
            pl.BlockSpec((1, 1, EPI_R, H), lambda i: (0, i // nb, i % nb, 0)),
            pl.BlockSpec((1, 1, NTILE, EPI_R),
                         lambda i: (0, i // nb, 0, i % nb)),
            pl.BlockSpec((H, H), lambda i: (0, 0)),
            pl.BlockSpec((3, H, H), lambda i: (0, 0, 0)),
            pl.BlockSpec((1, H), lambda i: (0, 0)),
            pl.BlockSpec((H, H), lambda i: (0, 0)),
            pl.BlockSpec((1, H), lambda i: (0, 0)),
        ],
        out_specs=pl.BlockSpec((EPI_R, H), lambda i: (i, 0)),
        out_shape=jax.ShapeDtypeStruct((NT, H), jnp.float32),
    )(table, sums_tt, cnts_tt, sums_ct, cnts_ct,
      w_h, w_agg, b_const, w_lin, b_lin)


def _split_edges(s, d):
    ss, dd = [], []
    for c in range(NCORE):
        valid = (d >= c * HALF) & (d < (c + 1) * HALF)
        ss.append(jnp.where(valid, s, IGN))
        dd.append(jnp.where(valid, d - c * HALF, IGN))
    return (jnp.stack(ss).reshape(NCORE, -1, CHUNK),
            jnp.stack(dd).reshape(NCORE, -1, CHUNK))


def kernel(x_target, x_context, lstm_t_Wih, lstm_t_Whh, lstm_t_bih, lstm_t_bhh,
           lstm_c_Wih, lstm_c_Whh, lstm_c_bih, lstm_c_bhh,
           dir_self_W, dir_self_b, dir_s2d_W, dir_s2d_b, dir_d2s_W, dir_d2s_b,
           ct_Wl, ct_bl, ct_Wr, lin_W, lin_b, edge_index_tt, edge_index_ct):
    f32 = jnp.float32
    i32 = jnp.int32
    p0 = E0_PAD - E_TT
    p2 = E2_PAD - E_CT
    zpad0 = jnp.zeros((p0,), i32)
    npad0 = jnp.full((p0,), NCORE * HALF, i32)
    s0, d0 = _split_edges(jnp.concatenate([edge_index_tt[0], zpad0]),
                          jnp.concatenate([edge_index_tt[1], npad0]))
    s1, d1 = _split_edges(jnp.concatenate([edge_index_tt[1], zpad0]),
                          jnp.concatenate([edge_index_tt[0], npad0]))
    s2, d2 = _split_edges(jnp.concatenate([edge_index_ct[0],
                                           jnp.zeros((p2,), i32)]),
                          jnp.concatenate([edge_index_ct[1],
                                           jnp.full((p2,), NCORE * HALF,
                                                    i32)]))

    table_t = _run_lstm(x_target, lstm_t_Wih, lstm_t_Whh, lstm_t_bih,
                        lstm_t_bhh)
    nch_tt = E0_PAD // (NTILE * CHUNK)
    sums_tt, cnts_tt = _run_sc_agg(table_t, (s0, d0, s1, d1),
                                   (nch_tt, nch_tt))
    table_c = _run_lstm(x_context, lstm_c_Wih, lstm_c_Whh, lstm_c_bih,
                        lstm_c_bhh)
    sums_ct, cnts_ct = _run_sc_agg(table_c, (s2, d2),
                                   (E2_PAD // (NTILE * CHUNK),))

    w_h = ((dir_self_W.T + ct_Wr.T) * 0.5).astype(f32)
    w_agg = jnp.stack([
        ((1.0 - ALPHA) * 0.5) * dir_s2d_W.T,
        (ALPHA * 0.5) * dir_d2s_W.T,
        0.5 * ct_Wl.T,
    ]).astype(f32)
    b_const = ((dir_self_b + (1.0 - ALPHA) * dir_s2d_b + ALPHA * dir_d2s_b
                + ct_bl) * 0.5)[None, :].astype(f32)
    return _run_epilogue(table_t, sums_tt, cnts_tt, sums_ct, cnts_ct,
                         w_h, w_agg, b_const,
                         lin_W.T.astype(f32), lin_b[None, :].astype(f32))


# speedup vs baseline: 7.7540x; 1.0035x over previous
import functools

import jax
import jax.numpy as jnp
from jax import lax
from jax.experimental import pallas as pl
from jax.experimental.pallas import tpu as pltpu
from jax.experimental.pallas import tpu_sc as plsc

H = 128
GW = 4 * H
NT = 10000
NC = 10000
SEQ = 8
ALPHA = 0.5
CHUNK = 128
LANES = 16
NTILE = 16
NCORE = 2
ACC_ROWS = 5120
HALF = ACC_ROWS
ROWS_PER_TILE = ACC_ROWS // NTILE
IGN = 1 << 30
E_TT = 320000
E_CT = 160000
E0_PAD = 327680
E2_PAD = 163840
LSTM_R = 2000
EPI_R = 1024


def _lstm_body(x_ref, wih_ref, whh_ref, b_ref, out_ref):
    x = x_ref[...]
    wih = wih_ref[...]
    whh = whh_ref[...]
    b = b_ref[...]

    def gates(g):
        i = jax.nn.sigmoid(g[:, 0:H])
        f = jax.nn.sigmoid(g[:, H:2 * H])
        gg = jnp.tanh(g[:, 2 * H:3 * H])
        o = jax.nn.sigmoid(g[:, 3 * H:4 * H])
        return i, f, gg, o

    g = x[:, 0:1] * wih + b
    i, f, gg, o = gates(g)
    c = i * gg
    h = o * jnp.tanh(c)
    for t in range(1, SEQ):
        g = x[:, t:t + 1] * wih + b
        g = g + jnp.dot(h, whh, preferred_element_type=jnp.float32)
        i, f, gg, o = gates(g)
        c = f * c + i * gg
        h = o * jnp.tanh(c)
    out_ref[...] = h


def _run_lstm(x, wih, whh, bih, bhh):
    n = x.shape[0]
    return pl.pallas_call(
        _lstm_body,
        grid=(n // LSTM_R,),
        in_specs=[
            pl.BlockSpec((LSTM_R, SEQ), lambda i: (i, 0)),
            pl.BlockSpec((1, GW), lambda i: (0, 0)),
            pl.BlockSpec((H, GW), lambda i: (0, 0)),
            pl.BlockSpec((1, GW), lambda i: (0, 0)),
        ],
        out_specs=pl.BlockSpec((LSTM_R, H), lambda i: (i, 0)),
        out_shape=jax.ShapeDtypeStruct((n, H), jnp.float32),
    )(x.astype(jnp.float32), wih.T, whh.T, (bih + bhh)[None, :])


def _sc_agg_body(n_aggs, chunk_counts, *refs):
    table_h = refs[0]
    edges = refs[1:1 + 2 * n_aggs]
    (zeros_h, zflat_h, sums_h, cnts_h,
     src_v, dst_v, rows_a, rows_b, hist_v, acc_sh, sem_a, sem_b) = \
        refs[1 + 2 * n_aggs:]
    cid = lax.axis_index("c")
    sid = lax.axis_index("s")
    base_row = sid * ROWS_PER_TILE

    def run(agg_i, srcs_h, dsts_h, ntile_chunks):
        pltpu.sync_copy(srcs_h.at[cid, pl.ds(sid * ntile_chunks, ntile_chunks)],
                        src_v.at[pl.ds(0, ntile_chunks)])
        pltpu.sync_copy(dsts_h.at[cid, pl.ds(sid * ntile_chunks, ntile_chunks)],
                        dst_v.at[pl.ds(0, ntile_chunks)])
        pltpu.sync_copy(zeros_h, acc_sh.at[pl.ds(base_row, CHUNK)])
        pltpu.sync_copy(zeros_h, acc_sh.at[pl.ds(base_row + CHUNK, CHUNK)])
        pltpu.sync_copy(zeros_h.at[pl.ds(0, ROWS_PER_TILE - 2 * CHUNK)],
                        acc_sh.at[pl.ds(base_row + 2 * CHUNK,
                                        ROWS_PER_TILE - 2 * CHUNK)])
        pltpu.sync_copy(zflat_h, hist_v)
        plsc.subcore_barrier()

        ones16 = jnp.ones((LANES,), jnp.float32)

        def gather(j, buf, sem):
            return pltpu.async_copy(
                table_h.at[plsc.Indices(src_v.at[j], ignored_value=IGN)],
                buf, sem)

        def scatter(j, buf):
            pltpu.sync_copy(
                buf,
                acc_sh.at[plsc.Indices(dst_v.at[j], ignored_value=IGN)],
                add=True)
            for k in range(CHUNK // LANES):
                d = dst_v[j, pl.ds(k * LANES, LANES)]
                plsc.addupdate_scatter(hist_v, [d], ones16,
                                       mask=d < ACC_ROWS)

        def chunk2(i, carry):
            j = 2 * i
            desc_b = gather(j + 1, rows_b, sem_b)
            pltpu.make_async_copy(
                table_h.at[plsc.Indices(src_v.at[j], ignored_value=IGN)],
                rows_a, sem_a).wait()
            scatter(j, rows_a)

            @pl.when(j + 2 < ntile_chunks)
            def _():
                gather(j + 2, rows_a, sem_a)

            desc_b.wait()
            scatter(j + 1, rows_b)
            return carry

        gather(0, rows_a, sem_a)
        lax.fori_loop(0, ntile_chunks // 2, chunk2, 0)
        plsc.subcore_barrier()
        pltpu.sync_copy(acc_sh.at[pl.ds(base_row, CHUNK)],
                        sums_h.at[agg_i, cid, pl.ds(base_row, CHUNK)])
        pltpu.sync_copy(acc_sh.at[pl.ds(base_row + CHUNK, CHUNK)],
                        sums_h.at[agg_i, cid, pl.ds(base_row + CHUNK, CHUNK)])
        pltpu.sync_copy(
            acc_sh.at[pl.ds(base_row + 2 * CHUNK, ROWS_PER_TILE - 2 * CHUNK)],
            sums_h.at[agg_i, cid, pl.ds(base_row + 2 * CHUNK,
                                        ROWS_PER_TILE - 2 * CHUNK)])
        pltpu.sync_copy(hist_v, cnts_h.at[agg_i, cid, sid])

    for a in range(n_aggs):
        run(a, edges[2 * a], edges[2 * a + 1], chunk_counts[a])


def _run_sc_agg(table, edges, chunk_counts):
    n_aggs = len(chunk_counts)
    zeros_chunk = jnp.zeros((CHUNK, H), jnp.float32)
    zeros_flat = jnp.zeros((ACC_ROWS,), jnp.float32)
    nch = max(chunk_counts)
    return pl.kernel(
        functools.partial(_sc_agg_body, n_aggs, tuple(chunk_counts)),
        out_type=(
            jax.ShapeDtypeStruct((n_aggs, NCORE, ACC_ROWS, H), jnp.float32),
            jax.ShapeDtypeStruct((n_aggs, NCORE, NTILE, ACC_ROWS),
                                 jnp.float32),
        ),
        mesh=plsc.VectorSubcoreMesh(core_axis_name="c", subcore_axis_name="s"),
        compiler_params=pltpu.CompilerParams(needs_layout_passes=False),
        scratch_types=[
            pltpu.VMEM((nch, CHUNK), jnp.int32),
            pltpu.VMEM((nch, CHUNK), jnp.int32),
            pltpu.VMEM((CHUNK, H), jnp.float32),
            pltpu.VMEM((CHUNK, H), jnp.float32),
            pltpu.VMEM((ACC_ROWS,), jnp.float32),
            pltpu.VMEM_SHARED((ACC_ROWS, H), jnp.float32),
            pltpu.SemaphoreType.DMA,
            pltpu.SemaphoreType.DMA,
        ],
    )(table, *edges, zeros_chunk, zeros_flat)


def _epi_body(t_ref, stt_ref, ctt_ref, sct_ref, cct_ref,
              wh_ref, wa_ref, bc_ref, wl_ref, bl_ref, out_ref):
    ht = t_ref[...]
    pre = jnp.dot(ht, wh_ref[...], preferred_element_type=jnp.float32) + bc_ref[...]
    parts = [(stt_ref[0, 0], ctt_ref[0, 0]), (stt_ref[1, 0], ctt_ref[1, 0]),
             (sct_ref[0, 0], cct_ref[0, 0])]
    for a, (s, craw) in enumerate(parts):
        cnt = jnp.maximum(jnp.sum(craw, axis=0), 1.0)
        pre = pre + jnp.dot(s / cnt[:, None], wa_ref[a],
                            preferred_element_type=jnp.float32)
    act = jnp.maximum(pre + ht, 0.0)
    out_ref[...] = (jnp.dot(act, wl_ref[...], preferred_element_type=jnp.float32)
                    + bl_ref[...])


def _run_epilogue(table, sums_tt, cnts_tt, sums_ct, cnts_ct,
                  w_h, w_agg, b_const, w_lin, b_lin):
    grid = (NT + EPI_R - 1) // EPI_R
    nb = HALF // EPI_R
    return pl.pallas_call(
        _epi_body,
        grid=(grid,),
        in_specs=[
            pl.BlockSpec((EPI_R, H), lambda i: (i, 0)),
            pl.BlockSpec((2, 1, EPI_R, H), lambda i: (0, i // nb, i % nb, 0)),
            pl.BlockSpec((2, 1, NTILE, EPI_R),
                         lambda i: (0, i // nb, 0, i % nb)),
            pl.BlockSpec((1, 1, EPI_R, H), lambda i: (0, i // nb, i % nb, 0)),
            pl.BlockSpec((1, 1, NTILE, EPI_R),
                         lambda i: (0, i // nb, 0, i % nb)),
            pl.BlockSpec((H, H), lambda i: (0, 0)),
            pl.BlockSpec((3, H, H), lambda i: (0, 0, 0)),
            pl.BlockSpec((1, H), lambda i: (0, 0)),
            pl.BlockSpec((H, H), lambda i: (0, 0)),
            pl.BlockSpec((1, H), lambda i: (0, 0)),
        ],
        out_specs=pl.BlockSpec((EPI_R, H), lambda i: (i, 0)),
        out_shape=jax.ShapeDtypeStruct((NT, H), jnp.float32),
    )(table, sums_tt, cnts_tt, sums_ct, cnts_ct,
      w_h, w_agg, b_const, w_lin, b_lin)


def _split_edges(s, d):
    ss, dd = [], []
    for c in range(NCORE):
        valid = (d >= c * HALF) & (d < (c + 1) * HALF)
        ss.append(jnp.where(valid, s, IGN))
        dd.append(jnp.where(valid, d - c * HALF, IGN))
    return (jnp.stack(ss).reshape(NCORE, -1, CHUNK),
            jnp.stack(dd).reshape(NCORE, -1, CHUNK))


def kernel(x_target, x_context, lstm_t_Wih, lstm_t_Whh, lstm_t_bih, lstm_t_bhh,
           lstm_c_Wih, lstm_c_Whh, lstm_c_bih, lstm_c_bhh,
           dir_self_W, dir_self_b, dir_s2d_W, dir_s2d_b, dir_d2s_W, dir_d2s_b,
           ct_Wl, ct_bl, ct_Wr, lin_W, lin_b, edge_index_tt, edge_index_ct):
    f32 = jnp.float32
    i32 = jnp.int32
    p0 = E0_PAD - E_TT
    p2 = E2_PAD - E_CT
    zpad0 = jnp.zeros((p0,), i32)
    npad0 = jnp.full((p0,), NCORE * HALF, i32)
    s0, d0 = _split_edges(jnp.concatenate([edge_index_tt[0], zpad0]),
                          jnp.concatenate([edge_index_tt[1], npad0]))
    s1, d1 = _split_edges(jnp.concatenate([edge_index_tt[1], zpad0]),
                          jnp.concatenate([edge_index_tt[0], npad0]))
    s2, d2 = _split_edges(jnp.concatenate([edge_index_ct[0],
                                           jnp.zeros((p2,), i32)]),
                          jnp.concatenate([edge_index_ct[1],
                                           jnp.full((p2,), NCORE * HALF,
                                                    i32)]))

    table_t = _run_lstm(x_target, lstm_t_Wih, lstm_t_Whh, lstm_t_bih,
                        lstm_t_bhh)
    nch_tt = E0_PAD // (NTILE * CHUNK)
    sums_tt, cnts_tt = _run_sc_agg(table_t, (s0, d0, s1, d1),
                                   (nch_tt, nch_tt))
    table_c = _run_lstm(x_context, lstm_c_Wih, lstm_c_Whh, lstm_c_bih,
                        lstm_c_bhh)
    sums_ct, cnts_ct = _run_sc_agg(table_c, (s2, d2),
                                   (E2_PAD // (NTILE * CHUNK),))

    w_h = ((dir_self_W.T + ct_Wr.T) * 0.5).astype(f32)
    w_agg = jnp.stack([
        ((1.0 - ALPHA) * 0.5) * dir_s2d_W.T,
        (ALPHA * 0.5) * dir_d2s_W.T,
        0.5 * ct_Wl.T,
    ]).astype(f32)
    b_const = ((dir_self_b + (1.0 - ALPHA) * dir_s2d_b + ALPHA * dir_d2s_b
                + ct_bl) * 0.5)[None, :].astype(f32)
    return _run_epilogue(table_t, sums_tt, cnts_tt, sums_ct, cnts_ct,
                         w_h, w_agg, b_const,
                         lin_W.T.astype(f32), lin_b[None, :].astype(f32))


# speedup vs baseline: 7.7969x; 1.0055x over previous
import functools

import jax
import jax.numpy as jnp
from jax import lax
from jax.experimental import pallas as pl
from jax.experimental.pallas import tpu as pltpu
from jax.experimental.pallas import tpu_sc as plsc

H = 128
GW = 4 * H
NT = 10000
NC = 10000
SEQ = 8
ALPHA = 0.5
CHUNK = 128
LANES = 16
NTILE = 16
NCORE = 2
ACC_ROWS = 5120
HALF = ACC_ROWS
ROWS_PER_TILE = ACC_ROWS // NTILE
IGN = 1 << 30
E_TT = 320000
E_CT = 160000
E0_PAD = 327680
E2_PAD = 163840
LSTM_R = 2000
EPI_R = 1024


def _lstm_body(x_ref, wih_ref, whh_ref, b_ref, out_ref):
    x = x_ref[...]
    wih = wih_ref[...]
    whh = whh_ref[...]
    b = b_ref[...]

    def gates(g):
        i = jax.nn.sigmoid(g[:, 0:H])
        f = jax.nn.sigmoid(g[:, H:2 * H])
        gg = jnp.tanh(g[:, 2 * H:3 * H])
        o = jax.nn.sigmoid(g[:, 3 * H:4 * H])
        return i, f, gg, o

    g = x[:, 0:1] * wih + b
    i, f, gg, o = gates(g)
    c = i * gg
    h = o * jnp.tanh(c)
    for t in range(1, SEQ):
        g = x[:, t:t + 1] * wih + b
        g = g + jnp.dot(h, whh, preferred_element_type=jnp.float32)
        i, f, gg, o = gates(g)
        c = f * c + i * gg
        h = o * jnp.tanh(c)
    out_ref[...] = h


def _run_lstm(x, wih, whh, bih, bhh):
    n = x.shape[0]
    return pl.pallas_call(
        _lstm_body,
        grid=(n // LSTM_R,),
        in_specs=[
            pl.BlockSpec((LSTM_R, SEQ), lambda i: (i, 0)),
            pl.BlockSpec((1, GW), lambda i: (0, 0)),
            pl.BlockSpec((H, GW), lambda i: (0, 0)),
            pl.BlockSpec((1, GW), lambda i: (0, 0)),
        ],
        out_specs=pl.BlockSpec((LSTM_R, H), lambda i: (i, 0)),
        out_shape=jax.ShapeDtypeStruct((n, H), jnp.float32),
    )(x.astype(jnp.float32), wih.T, whh.T, (bih + bhh)[None, :])


def _sc_agg_body(n_aggs, chunk_counts, *refs):
    table_h = refs[0]
    edges = refs[1:1 + 2 * n_aggs]
    (zeros_h, zflat_h, sums_h, cnts_h,
     src_v, dst_v, rows_a, rows_b, hist_v, acc_sh, sem_a, sem_b) = \
        refs[1 + 2 * n_aggs:]
    cid = lax.axis_index("c")
    sid = lax.axis_index("s")
    base_row = sid * ROWS_PER_TILE

    def run(agg_i, srcs_h, dsts_h, ntile_chunks):
        pltpu.sync_copy(srcs_h.at[pl.ds(sid * ntile_chunks, ntile_chunks)],
                        src_v.at[pl.ds(0, ntile_chunks)])
        pltpu.sync_copy(dsts_h.at[pl.ds(sid * ntile_chunks, ntile_chunks)],
                        dst_v.at[pl.ds(0, ntile_chunks)])
        pltpu.sync_copy(zeros_h, acc_sh.at[pl.ds(base_row, CHUNK)])
        pltpu.sync_copy(zeros_h, acc_sh.at[pl.ds(base_row + CHUNK, CHUNK)])
        pltpu.sync_copy(zeros_h.at[pl.ds(0, ROWS_PER_TILE - 2 * CHUNK)],
                        acc_sh.at[pl.ds(base_row + 2 * CHUNK,
                                        ROWS_PER_TILE - 2 * CHUNK)])
        pltpu.sync_copy(zflat_h, hist_v)
        plsc.subcore_barrier()

        ones16 = jnp.ones((LANES,), jnp.float32)
        lo = cid * HALF

        def transform(j):
            for k in range(CHUNK // LANES):
                o = k * LANES
                d = dst_v[j, pl.ds(o, LANES)]
                s = src_v[j, pl.ds(o, LANES)]
                valid = (d >= lo) & (d < lo + HALF)
                dst_v[j, pl.ds(o, LANES)] = jnp.where(valid, d - lo, IGN)
                src_v[j, pl.ds(o, LANES)] = jnp.where(valid, s, IGN)

        def gather(j, buf, sem):
            return pltpu.async_copy(
                table_h.at[plsc.Indices(src_v.at[j], ignored_value=IGN)],
                buf, sem)

        def scatter(j, buf):
            pltpu.sync_copy(
                buf,
                acc_sh.at[plsc.Indices(dst_v.at[j], ignored_value=IGN)],
                add=True)
            for k in range(CHUNK // LANES):
                d = dst_v[j, pl.ds(k * LANES, LANES)]
                plsc.addupdate_scatter(hist_v, [d], ones16,
                                       mask=d < ACC_ROWS)

        def chunk2(i, carry):
            j = 2 * i
            desc_b = gather(j + 1, rows_b, sem_b)

            @pl.when(j + 2 < ntile_chunks)
            def _():
                transform(j + 2)
                transform(j + 3)

            pltpu.make_async_copy(
                table_h.at[plsc.Indices(src_v.at[j], ignored_value=IGN)],
                rows_a, sem_a).wait()
            scatter(j, rows_a)

            @pl.when(j + 2 < ntile_chunks)
            def _():
                gather(j + 2, rows_a, sem_a)

            desc_b.wait()
            scatter(j + 1, rows_b)
            return carry

        transform(0)
        transform(1)
        gather(0, rows_a, sem_a)
        lax.fori_loop(0, ntile_chunks // 2, chunk2, 0)
        plsc.subcore_barrier()
        pltpu.sync_copy(acc_sh.at[pl.ds(base_row, CHUNK)],
                        sums_h.at[agg_i, cid, pl.ds(base_row, CHUNK)])
        pltpu.sync_copy(acc_sh.at[pl.ds(base_row + CHUNK, CHUNK)],
                        sums_h.at[agg_i, cid, pl.ds(base_row + CHUNK, CHUNK)])
        pltpu.sync_copy(
            acc_sh.at[pl.ds(base_row + 2 * CHUNK, ROWS_PER_TILE - 2 * CHUNK)],
            sums_h.at[agg_i, cid, pl.ds(base_row + 2 * CHUNK,
                                        ROWS_PER_TILE - 2 * CHUNK)])
        pltpu.sync_copy(hist_v, cnts_h.at[agg_i, cid, sid])

    for a in range(n_aggs):
        run(a, edges[2 * a], edges[2 * a + 1], chunk_counts[a])


def _run_sc_agg(table, edges, chunk_counts):
    n_aggs = len(chunk_counts)
    zeros_chunk = jnp.zeros((CHUNK, H), jnp.float32)
    zeros_flat = jnp.zeros((ACC_ROWS,), jnp.float32)
    nch = max(chunk_counts)
    return pl.kernel(
        functools.partial(_sc_agg_body, n_aggs, tuple(chunk_counts)),
        out_type=(
            jax.ShapeDtypeStruct((n_aggs, NCORE, ACC_ROWS, H), jnp.float32),
            jax.ShapeDtypeStruct((n_aggs, NCORE, NTILE, ACC_ROWS),
                                 jnp.float32),
        ),
        mesh=plsc.VectorSubcoreMesh(core_axis_name="c", subcore_axis_name="s"),
        compiler_params=pltpu.CompilerParams(needs_layout_passes=False),
        scratch_types=[
            pltpu.VMEM((nch, CHUNK), jnp.int32),
            pltpu.VMEM((nch, CHUNK), jnp.int32),
            pltpu.VMEM((CHUNK, H), jnp.float32),
            pltpu.VMEM((CHUNK, H), jnp.float32),
            pltpu.VMEM((ACC_ROWS,), jnp.float32),
            pltpu.VMEM_SHARED((ACC_ROWS, H), jnp.float32),
            pltpu.SemaphoreType.DMA,
            pltpu.SemaphoreType.DMA,
        ],
    )(table, *edges, zeros_chunk, zeros_flat)


def _epi_body(t_ref, stt_ref, ctt_ref, sct_ref, cct_ref,
              wh_ref, wa_ref, bc_ref, wl_ref, bl_ref, out_ref):
    ht = t_ref[...]
    pre = jnp.dot(ht, wh_ref[...], preferred_element_type=jnp.float32) + bc_ref[...]
    parts = [(stt_ref[0, 0], ctt_ref[0, 0]), (stt_ref[1, 0], ctt_ref[1, 0]),
             (sct_ref[0, 0], cct_ref[0, 0])]
    for a, (s, craw) in enumerate(parts):
        cnt = jnp.maximum(jnp.sum(craw, axis=0), 1.0)
        pre = pre + jnp.dot(s / cnt[:, None], wa_ref[a],
                            preferred_element_type=jnp.float32)
    act = jnp.maximum(pre + ht, 0.0)
    out_ref[...] = (jnp.dot(act, wl_ref[...], preferred_element_type=jnp.float32)
                    + bl_ref[...])


def _run_epilogue(table, sums_tt, cnts_tt, sums_ct, cnts_ct,
                  w_h, w_agg, b_const, w_lin, b_lin):
    grid = (NT + EPI_R - 1) // EPI_R
    nb = HALF // EPI_R
    return pl.pallas_call(
        _epi_body,
        grid=(grid,),
        in_specs=[
            pl.BlockSpec((EPI_R, H), lambda i: (i, 0)),
            pl.BlockSpec((2, 1, EPI_R, H), lambda i: (0, i // nb, i % nb, 0)),
            pl.BlockSpec((2, 1, NTILE, EPI_R),
                         lambda i: (0, i // nb, 0, i % nb)),
            pl.BlockSpec((1, 1, EPI_R, H), lambda i: (0, i // nb, i % nb, 0)),
            pl.BlockSpec((1, 1, NTILE, EPI_R),
                         lambda i: (0, i // nb, 0, i % nb)),
            pl.BlockSpec((H, H), lambda i: (0, 0)),
            pl.BlockSpec((3, H, H), lambda i: (0, 0, 0)),
            pl.BlockSpec((1, H), lambda i: (0, 0)),
            pl.BlockSpec((H, H), lambda i: (0, 0)),
            pl.BlockSpec((1, H), lambda i: (0, 0)),
        ],
        out_specs=pl.BlockSpec((EPI_R, H), lambda i: (i, 0)),
        out_shape=jax.ShapeDtypeStruct((NT, H), jnp.float32),
    )(table, sums_tt, cnts_tt, sums_ct, cnts_ct,
      w_h, w_agg, b_const, w_lin, b_lin)


def kernel(x_target, x_context, lstm_t_Wih, lstm_t_Whh, lstm_t_bih, lstm_t_bhh,
           lstm_c_Wih, lstm_c_Whh, lstm_c_bih, lstm_c_bhh,
           dir_self_W, dir_self_b, dir_s2d_W, dir_s2d_b, dir_d2s_W, dir_d2s_b,
           ct_Wl, ct_bl, ct_Wr, lin_W, lin_b, edge_index_tt, edge_index_ct):
    f32 = jnp.float32
    i32 = jnp.int32
    pad0 = jnp.full((E0_PAD - E_TT,), NCORE * HALF, i32)
    pad2 = jnp.full((E2_PAD - E_CT,), NCORE * HALF, i32)
    e_tt0 = jnp.concatenate([edge_index_tt[0], pad0]).reshape(-1, CHUNK)
    e_tt1 = jnp.concatenate([edge_index_tt[1], pad0]).reshape(-1, CHUNK)
    e_ct0 = jnp.concatenate([edge_index_ct[0], pad2]).reshape(-1, CHUNK)
    e_ct1 = jnp.concatenate([edge_index_ct[1], pad2]).reshape(-1, CHUNK)

    table_t = _run_lstm(x_target, lstm_t_Wih, lstm_t_Whh, lstm_t_bih,
                        lstm_t_bhh)
    nch_tt = E0_PAD // (NTILE * CHUNK)
    sums_tt, cnts_tt = _run_sc_agg(table_t, (e_tt0, e_tt1, e_tt1, e_tt0),
                                   (nch_tt, nch_tt))
    table_c = _run_lstm(x_context, lstm_c_Wih, lstm_c_Whh, lstm_c_bih,
                        lstm_c_bhh)
    sums_ct, cnts_ct = _run_sc_agg(table_c, (e_ct0, e_ct1),
                                   (E2_PAD // (NTILE * CHUNK),))

    w_h = ((dir_self_W.T + ct_Wr.T) * 0.5).astype(f32)
    w_agg = jnp.stack([
        ((1.0 - ALPHA) * 0.5) * dir_s2d_W.T,
        (ALPHA * 0.5) * dir_d2s_W.T,
        0.5 * ct_Wl.T,
    ]).astype(f32)
    b_const = ((dir_self_b + (1.0 - ALPHA) * dir_s2d_b + ALPHA * dir_d2s_b
                + ct_bl) * 0.5)[None, :].astype(f32)
    return _run_epilogue(table_t, sums_tt, cnts_tt, sums_ct, cnts_ct,
                         w_h, w_agg, b_const,
                         lin_W.T.astype(f32), lin_b[None, :].astype(f32))


# speedup vs baseline: 7.8662x; 1.0089x over previous
import functools

import jax
import jax.numpy as jnp
from jax import lax
from jax.experimental import pallas as pl
from jax.experimental.pallas import tpu as pltpu
from jax.experimental.pallas import tpu_sc as plsc

H = 128
GW = 4 * H
NT = 10000
NC = 10000
SEQ = 8
ALPHA = 0.5
CHUNK = 128
LANES = 16
NTILE = 16
NCORE = 2
ACC_ROWS = 5120
HALF = ACC_ROWS
ROWS_PER_TILE = ACC_ROWS // NTILE
IGN = 1 << 30
E_TT = 320000
E_CT = 160000
E0_PAD = 327680
E2_PAD = 163840
LSTM_R = 2000
EPI_R = 1024


def _lstm_body(x_ref, wih_ref, whh_ref, bih_ref, bhh_ref, out_ref):
    x = x_ref[...]
    wih = wih_ref[...]
    whh = whh_ref[...]
    b = bih_ref[...] + bhh_ref[...]

    def gates(g):
        i = jax.nn.sigmoid(g[:, 0:H])
        f = jax.nn.sigmoid(g[:, H:2 * H])
        gg = jnp.tanh(g[:, 2 * H:3 * H])
        o = jax.nn.sigmoid(g[:, 3 * H:4 * H])
        return i, f, gg, o

    g = x[:, 0:1] * wih + b
    i, f, gg, o = gates(g)
    c = i * gg
    h = o * jnp.tanh(c)
    for t in range(1, SEQ):
        g = x[:, t:t + 1] * wih + b
        g = g + jnp.dot(h, whh, preferred_element_type=jnp.float32)
        i, f, gg, o = gates(g)
        c = f * c + i * gg
        h = o * jnp.tanh(c)
    out_ref[...] = h


def _run_lstm(x, wih, whh, bih, bhh):
    n = x.shape[0]
    return pl.pallas_call(
        _lstm_body,
        grid=(n // LSTM_R,),
        in_specs=[
            pl.BlockSpec((LSTM_R, SEQ), lambda i: (i, 0)),
            pl.BlockSpec((1, GW), lambda i: (0, 0)),
            pl.BlockSpec((H, GW), lambda i: (0, 0)),
            pl.BlockSpec((1, GW), lambda i: (0, 0)),
            pl.BlockSpec((1, GW), lambda i: (0, 0)),
        ],
        out_specs=pl.BlockSpec((LSTM_R, H), lambda i: (i, 0)),
        out_shape=jax.ShapeDtypeStruct((n, H), jnp.float32),
    )(x.astype(jnp.float32), wih.T, whh.T, bih[None, :], bhh[None, :])


def _sc_agg_body(n_aggs, chunk_counts, *refs):
    table_h = refs[0]
    edges = refs[1:1 + 2 * n_aggs]
    (zeros_h, zflat_h, sums_h, cnts_h,
     src_v, dst_v, rows_a, rows_b, hist_v, acc_sh, sem_a, sem_b) = \
        refs[1 + 2 * n_aggs:]
    cid = lax.axis_index("c")
    sid = lax.axis_index("s")
    base_row = sid * ROWS_PER_TILE

    def run(agg_i, srcs_h, dsts_h, ntile_chunks):
        pltpu.sync_copy(srcs_h.at[pl.ds(sid * ntile_chunks, ntile_chunks)],
                        src_v.at[pl.ds(0, ntile_chunks)])
        pltpu.sync_copy(dsts_h.at[pl.ds(sid * ntile_chunks, ntile_chunks)],
                        dst_v.at[pl.ds(0, ntile_chunks)])
        pltpu.sync_copy(zeros_h, acc_sh.at[pl.ds(base_row, CHUNK)])
        pltpu.sync_copy(zeros_h, acc_sh.at[pl.ds(base_row + CHUNK, CHUNK)])
        pltpu.sync_copy(zeros_h.at[pl.ds(0, ROWS_PER_TILE - 2 * CHUNK)],
                        acc_sh.at[pl.ds(base_row + 2 * CHUNK,
                                        ROWS_PER_TILE - 2 * CHUNK)])
        pltpu.sync_copy(zflat_h, hist_v)
        plsc.subcore_barrier()

        ones16 = jnp.ones((LANES,), jnp.float32)
        lo = cid * HALF

        def transform(j):
            for k in range(CHUNK // LANES):
                o = k * LANES
                d = dst_v[j, pl.ds(o, LANES)]
                s = src_v[j, pl.ds(o, LANES)]
                valid = (d >= lo) & (d < lo + HALF)
                dst_v[j, pl.ds(o, LANES)] = jnp.where(valid, d - lo, IGN)
                src_v[j, pl.ds(o, LANES)] = jnp.where(valid, s, IGN)

        def gather(j, buf, sem):
            return pltpu.async_copy(
                table_h.at[plsc.Indices(src_v.at[j], ignored_value=IGN)],
                buf, sem)

        def scatter(j, buf):
            pltpu.sync_copy(
                buf,
                acc_sh.at[plsc.Indices(dst_v.at[j], ignored_value=IGN)],
                add=True)
            for k in range(CHUNK // LANES):
                d = dst_v[j, pl.ds(k * LANES, LANES)]
                plsc.addupdate_scatter(hist_v, [d], ones16,
                                       mask=d < ACC_ROWS)

        def chunk2(i, carry):
            j = 2 * i
            desc_b = gather(j + 1, rows_b, sem_b)

            @pl.when(j + 2 < ntile_chunks)
            def _():
                transform(j + 2)
                transform(j + 3)

            pltpu.make_async_copy(
                table_h.at[plsc.Indices(src_v.at[j], ignored_value=IGN)],
                rows_a, sem_a).wait()
            scatter(j, rows_a)

            @pl.when(j + 2 < ntile_chunks)
            def _():
                gather(j + 2, rows_a, sem_a)

            desc_b.wait()
            scatter(j + 1, rows_b)
            return carry

        transform(0)
        transform(1)
        gather(0, rows_a, sem_a)
        lax.fori_loop(0, ntile_chunks // 2, chunk2, 0)
        plsc.subcore_barrier()
        pltpu.sync_copy(acc_sh.at[pl.ds(base_row, CHUNK)],
                        sums_h.at[agg_i, cid, pl.ds(base_row, CHUNK)])
        pltpu.sync_copy(acc_sh.at[pl.ds(base_row + CHUNK, CHUNK)],
                        sums_h.at[agg_i, cid, pl.ds(base_row + CHUNK, CHUNK)])
        pltpu.sync_copy(
            acc_sh.at[pl.ds(base_row + 2 * CHUNK, ROWS_PER_TILE - 2 * CHUNK)],
            sums_h.at[agg_i, cid, pl.ds(base_row + 2 * CHUNK,
                                        ROWS_PER_TILE - 2 * CHUNK)])
        pltpu.sync_copy(hist_v, cnts_h.at[agg_i, cid, sid])

    for a in range(n_aggs):
        run(a, edges[2 * a], edges[2 * a + 1], chunk_counts[a])


def _run_sc_agg(table, edges, chunk_counts):
    n_aggs = len(chunk_counts)
    zeros_chunk = jnp.zeros((CHUNK, H), jnp.float32)
    zeros_flat = jnp.zeros((ACC_ROWS,), jnp.float32)
    nch = max(chunk_counts)
    return pl.kernel(
        functools.partial(_sc_agg_body, n_aggs, tuple(chunk_counts)),
        out_type=(
            jax.ShapeDtypeStruct((n_aggs, NCORE, ACC_ROWS, H), jnp.float32),
            jax.ShapeDtypeStruct((n_aggs, NCORE, NTILE, ACC_ROWS),
                                 jnp.float32),
        ),
        mesh=plsc.VectorSubcoreMesh(core_axis_name="c", subcore_axis_name="s"),
        compiler_params=pltpu.CompilerParams(needs_layout_passes=False),
        scratch_types=[
            pltpu.VMEM((nch, CHUNK), jnp.int32),
            pltpu.VMEM((nch, CHUNK), jnp.int32),
            pltpu.VMEM((CHUNK, H), jnp.float32),
            pltpu.VMEM((CHUNK, H), jnp.float32),
            pltpu.VMEM((ACC_ROWS,), jnp.float32),
            pltpu.VMEM_SHARED((ACC_ROWS, H), jnp.float32),
            pltpu.SemaphoreType.DMA,
            pltpu.SemaphoreType.DMA,
        ],
    )(table, *edges, zeros_chunk, zeros_flat)


def _epi_body(t_ref, stt_ref, ctt_ref, sct_ref, cct_ref,
              wh_ref, wa_ref, bc_ref, wl_ref, bl_ref, out_ref):
    ht = t_ref[...]
    pre = jnp.dot(ht, wh_ref[...], preferred_element_type=jnp.float32) + bc_ref[...]
    parts = [(stt_ref[0, 0], ctt_ref[0, 0]), (stt_ref[1, 0], ctt_ref[1, 0]),
             (sct_ref[0, 0], cct_ref[0, 0])]
    for a, (s, craw) in enumerate(parts):
        cnt = jnp.maximum(jnp.sum(craw, axis=0), 1.0)
        pre = pre + jnp.dot(s / cnt[:, None], wa_ref[a],
                            preferred_element_type=jnp.float32)
    act = jnp.maximum(pre + ht, 0.0)
    out_ref[...] = (jnp.dot(act, wl_ref[...], preferred_element_type=jnp.float32)
                    + bl_ref[...])


def _run_epilogue(table, sums_tt, cnts_tt, sums_ct, cnts_ct,
                  w_h, w_agg, b_const, w_lin, b_lin):
    grid = (NT + EPI_R - 1) // EPI_R
    nb = HALF // EPI_R
    return pl.pallas_call(
        _epi_body,
        grid=(grid,),
        in_specs=[
            pl.BlockSpec((EPI_R, H), lambda i: (i, 0)),
            pl.BlockSpec((2, 1, EPI_R, H), lambda i: (0, i // nb, i % nb, 0)),
            pl.BlockSpec((2, 1, NTILE, EPI_R),
                         lambda i: (0, i // nb, 0, i % nb)),
            pl.BlockSpec((1, 1, EPI_R, H), lambda i: (0, i // nb, i % nb, 0)),
            pl.BlockSpec((1, 1, NTILE, EPI_R),
                         lambda i: (0, i // nb, 0, i % nb)),
            pl.BlockSpec((H, H), lambda i: (0, 0)),
            pl.BlockSpec((3, H, H), lambda i: (0, 0, 0)),
            pl.BlockSpec((1, H), lambda i: (0, 0)),
            pl.BlockSpec((H, H), lambda i: (0, 0)),
            pl.BlockSpec((1, H), lambda i: (0, 0)),
        ],
        out_specs=pl.BlockSpec((EPI_R, H), lambda i: (i, 0)),
        out_shape=jax.ShapeDtypeStruct((NT, H), jnp.float32),
    )(table, sums_tt, cnts_tt, sums_ct, cnts_ct,
      w_h, w_agg, b_const, w_lin, b_lin)


def kernel(x_target, x_context, lstm_t_Wih, lstm_t_Whh, lstm_t_bih, lstm_t_bhh,
           lstm_c_Wih, lstm_c_Whh, lstm_c_bih, lstm_c_bhh,
           dir_self_W, dir_self_b, dir_s2d_W, dir_s2d_b, dir_d2s_W, dir_d2s_b,
           ct_Wl, ct_bl, ct_Wr, lin_W, lin_b, edge_index_tt, edge_index_ct):
    f32 = jnp.float32
    i32 = jnp.int32
    pad0 = jnp.full((E0_PAD - E_TT,), NCORE * HALF, i32)
    pad2 = jnp.full((E2_PAD - E_CT,), NCORE * HALF, i32)
    e_tt0 = jnp.concatenate([edge_index_tt[0], pad0]).reshape(-1, CHUNK)
    e_tt1 = jnp.concatenate([edge_index_tt[1], pad0]).reshape(-1, CHUNK)
    e_ct0 = jnp.concatenate([edge_index_ct[0], pad2]).reshape(-1, CHUNK)
    e_ct1 = jnp.concatenate([edge_index_ct[1], pad2]).reshape(-1, CHUNK)

    table_t = _run_lstm(x_target, lstm_t_Wih, lstm_t_Whh, lstm_t_bih,
                        lstm_t_bhh)
    nch_tt = E0_PAD // (NTILE * CHUNK)
    sums_tt, cnts_tt = _run_sc_agg(table_t, (e_tt0, e_tt1, e_tt1, e_tt0),
                                   (nch_tt, nch_tt))
    table_c = _run_lstm(x_context, lstm_c_Wih, lstm_c_Whh, lstm_c_bih,
                        lstm_c_bhh)
    sums_ct, cnts_ct = _run_sc_agg(table_c, (e_ct0, e_ct1),
                                   (E2_PAD // (NTILE * CHUNK),))

    w_h = ((dir_self_W.T + ct_Wr.T) * 0.5).astype(f32)
    w_agg = jnp.stack([
        ((1.0 - ALPHA) * 0.5) * dir_s2d_W.T,
        (ALPHA * 0.5) * dir_d2s_W.T,
        0.5 * ct_Wl.T,
    ]).astype(f32)
    b_const = ((dir_self_b + (1.0 - ALPHA) * dir_s2d_b + ALPHA * dir_d2s_b
                + ct_bl) * 0.5)[None, :].astype(f32)
    return _run_epilogue(table_t, sums_tt, cnts_tt, sums_ct, cnts_ct,
                         w_h, w_agg, b_const,
                         lin_W.T.astype(f32), lin_b[None, :].astype(f32))
